# baseline (speedup 1.0000x reference)
import functools

import jax
import jax.numpy as jnp
from jax import lax
from jax.experimental import pallas as pl
from jax.experimental.pallas import tpu as pltpu
from jax.experimental.pallas import tpu_sc as plsc

_CROP_H = 7
_CROP_W = 7
_S = _CROP_H * _CROP_W

_B, _C, _H, _W = 4, 256, 224, 224
_HW = _H * _W
_NBOX = 2000
_NP = 2048
_SP = 64
_P = _NP * _SP

_NC, _NS = 2, 16
_NW = _NC * _NS
_PPW = _P // _NW
_G = 32

_HWB = 3584


def _transpose_in_body(x_ref, o_ref):
    o_ref[0] = x_ref[0].T


def _make_table(image):
    img3 = image.reshape(_B, _C, _HW)
    t = pl.pallas_call(
        _transpose_in_body,
        grid=(_B, _HW // _HWB),
        in_specs=[pl.BlockSpec((1, _C, _HWB), lambda b, h: (b, 0, h))],
        out_specs=pl.BlockSpec((1, _HWB, _C), lambda b, h: (b, h, 0)),
        out_shape=jax.ShapeDtypeStruct((_B, _HW, _C), jnp.float32),
    )(img3)
    return t.reshape(_B * _HW, _C)


def _coords_body(boxes_ref, bidx_ref, r00, r01, r10, r11, w00, w01, w10, w11):
    boxes = boxes_ref[...]
    y1 = boxes[:, 0:1]
    x1 = boxes[:, 1:2]
    y2 = boxes[:, 2:3]
    x2 = boxes[:, 3:4]
    s = lax.broadcasted_iota(jnp.int32, (_NP, _SP), 1)
    i = (s // _CROP_W).astype(jnp.float32)
    j = (s % _CROP_W).astype(jnp.float32)
    valid = s < _S
    ys = y1 * (_H - 1) + i * ((y2 - y1) * (_H - 1) / (_CROP_H - 1))
    xs = x1 * (_W - 1) + j * ((x2 - x1) * (_W - 1) / (_CROP_W - 1))
    oy = (ys < 0) | (ys > (_H - 1))
    ox = (xs < 0) | (xs > (_W - 1))
    y0f = jnp.floor(ys)
    x0f = jnp.floor(xs)
    yl = ys - y0f
    xl = xs - x0f
    y0i = jnp.clip(y0f, 0, _H - 1).astype(jnp.int32)
    y1i = jnp.clip(y0f + 1, 0, _H - 1).astype(jnp.int32)
    x0i = jnp.clip(x0f, 0, _W - 1).astype(jnp.int32)
    x1i = jnp.clip(x0f + 1, 0, _W - 1).astype(jnp.int32)
    base = bidx_ref[:, 0:1] * _HW
    r00[...] = base + y0i * _W + x0i
    r01[...] = base + y0i * _W + x1i
    r10[...] = base + y1i * _W + x0i
    r11[...] = base + y1i * _W + x1i
    vf = jnp.where(valid & ~oy & ~ox, 1.0, 0.0).astype(jnp.float32)
    w00[...] = (1.0 - yl) * (1.0 - xl) * vf
    w01[...] = (1.0 - yl) * xl * vf
    w10[...] = yl * (1.0 - xl) * vf
    w11[...] = yl * xl * vf


def _coords(boxes_p, bidx_p):
    i32 = jax.ShapeDtypeStruct((_NP, _SP), jnp.int32)
    f32 = jax.ShapeDtypeStruct((_NP, _SP), jnp.float32)
    return pl.pallas_call(
        _coords_body,
        out_shape=(i32, i32, i32, i32, f32, f32, f32, f32),
    )(boxes_p, bidx_p)


_MESH = plsc.VectorSubcoreMesh(
    core_axis_name="c", subcore_axis_name="s", num_cores=_NC, num_subcores=_NS
)


def _sc_gather(table, r00, r01, r10, r11, w00, w01, w10, w11):
    @functools.partial(
        pl.kernel,
        out_type=jax.ShapeDtypeStruct((_P, _C), jnp.float32),
        mesh=_MESH,
        scratch_types=[
            pltpu.VMEM((_PPW,), jnp.int32),
            pltpu.VMEM((_PPW,), jnp.int32),
            pltpu.VMEM((_PPW,), jnp.int32),
            pltpu.VMEM((_PPW,), jnp.int32),
            pltpu.VMEM((_PPW,), jnp.float32),
            pltpu.VMEM((_PPW,), jnp.float32),
            pltpu.VMEM((_PPW,), jnp.float32),
            pltpu.VMEM((_PPW,), jnp.float32),
            pltpu.VMEM((_G, _C), jnp.float32),
            pltpu.VMEM((_G, _C), jnp.float32),
            pltpu.VMEM((_G, _C), jnp.float32),
            pltpu.VMEM((_G, _C), jnp.float32),
            pltpu.VMEM((_G, _C), jnp.float32),
            pltpu.SemaphoreType.DMA,
            pltpu.SemaphoreType.DMA,
            pltpu.SemaphoreType.DMA,
            pltpu.SemaphoreType.DMA,
        ],
    )
    def k(table_h, r00_h, r01_h, r10_h, r11_h, w00_h, w01_h, w10_h, w11_h,
          out_h, i0v, i1v, i2v, i3v, w0v, w1v, w2v, w3v, v0, v1, v2, v3, ob,
          sem0, sem1, sem2, sem3):
        wid = lax.axis_index("s") * _NC + lax.axis_index("c")
        base = wid * _PPW
        pltpu.sync_copy(r00_h.at[pl.ds(base, _PPW)], i0v)
        pltpu.sync_copy(r01_h.at[pl.ds(base, _PPW)], i1v)
        pltpu.sync_copy(r10_h.at[pl.ds(base, _PPW)], i2v)
        pltpu.sync_copy(r11_h.at[pl.ds(base, _PPW)], i3v)
        pltpu.sync_copy(w00_h.at[pl.ds(base, _PPW)], w0v)
        pltpu.sync_copy(w01_h.at[pl.ds(base, _PPW)], w1v)
        pltpu.sync_copy(w10_h.at[pl.ds(base, _PPW)], w2v)
        pltpu.sync_copy(w11_h.at[pl.ds(base, _PPW)], w3v)

        def chunk_body(g, carry):
            off = g * _G
            c0 = pltpu.async_copy(table_h.at[i0v.at[pl.ds(off, _G)]], v0, sem0)
            c1 = pltpu.async_copy(table_h.at[i1v.at[pl.ds(off, _G)]], v1, sem1)
            c2 = pltpu.async_copy(table_h.at[i2v.at[pl.ds(off, _G)]], v2, sem2)
            c3 = pltpu.async_copy(table_h.at[i3v.at[pl.ds(off, _G)]], v3, sem3)
            c0.wait()
            c1.wait()
            c2.wait()
            c3.wait()

            def group_body(t, carry2):
                l0 = t * 16
                wv0 = w0v[pl.ds(off + l0, 16)]
                wv1 = w1v[pl.ds(off + l0, 16)]
                wv2 = w2v[pl.ds(off + l0, 16)]
                wv3 = w3v[pl.ds(off + l0, 16)]
                for m in range(16):
                    l = l0 + m
                    a0 = jnp.full((16,), wv0[m], jnp.float32)
                    a1 = jnp.full((16,), wv1[m], jnp.float32)
                    a2 = jnp.full((16,), wv2[m], jnp.float32)
                    a3 = jnp.full((16,), wv3[m], jnp.float32)
                    for cc in range(_C // 16):
                        sl = pl.ds(cc * 16, 16)
                        ob[l, sl] = (v0[l, sl] * a0 + v1[l, sl] * a1
                                     + v2[l, sl] * a2 + v3[l, sl] * a3)
                return carry2

            lax.fori_loop(0, _G // 16, group_body, 0)
            pltpu.sync_copy(ob, out_h.at[pl.ds(base + off, _G)])
            return carry

        lax.fori_loop(0, _PPW // _G, chunk_body, 0)

    return k(table, r00, r01, r10, r11, w00, w01, w10, w11)


_NBLK = 8


def _final_body(x_ref, o_ref):
    x = x_ref[...]
    t = jnp.transpose(x, (0, 2, 1))
    o_ref[...] = t[:, :, :_S]


def _final(out3):
    return pl.pallas_call(
        _final_body,
        grid=(_NBOX // _NBLK,),
        in_specs=[pl.BlockSpec((_NBLK, _SP, _C), lambda n: (n, 0, 0))],
        out_specs=pl.BlockSpec((_NBLK, _C, _S), lambda n: (n, 0, 0)),
        out_shape=jax.ShapeDtypeStruct((_NBOX, _C, _S), jnp.float32),
    )(out3)


def kernel(image, boxes, box_indices):
    table = _make_table(image)
    boxes_p = jnp.pad(boxes, ((0, _NP - _NBOX), (0, 0)))
    bidx_p = jnp.broadcast_to(
        jnp.pad(box_indices.astype(jnp.int32), (0, _NP - _NBOX))[:, None],
        (_NP, 128),
    )
    r00, r01, r10, r11, w00, w01, w10, w11 = _coords(boxes_p, bidx_p)
    flat_i = lambda a: a.reshape(_P)
    out_flat = _sc_gather(
        table,
        flat_i(r00), flat_i(r01), flat_i(r10), flat_i(r11),
        flat_i(w00), flat_i(w01), flat_i(w10), flat_i(w11),
    )
    out3 = out_flat.reshape(_NP, _SP, _C)
    out = _final(out3)
    return out.reshape(_NBOX, _C, _CROP_H, _CROP_W)
